# baseline (speedup 1.0000x reference)
import functools

import jax
import jax.numpy as jnp
from jax import lax
from jax.experimental import pallas as pl
from jax.experimental.pallas import tpu as pltpu
from jax.experimental.pallas import tpu_sc as plsc

NC, NS, L = 2, 16, 16
NW = NC * NS
B, H, D = 4096, 200, 64
RPW = B // NW
EPW = RPW * H
CH = 128
NCHUNK = EPW // CH
ACC_ROWS = NS * RPW
TRASH0 = ACC_ROWS


def _iota16():
    return lax.broadcasted_iota(jnp.int32, (L,), 0)


def _body(ids_hbm, mask_hbm, table_hbm, out_hbm,
          ids_v, mask_v, gbuf, dstb, inv_v, blk_v, acc_sh, sem):
    c = lax.axis_index("c")
    s = lax.axis_index("s")
    wid = c * NS + s
    ebase = wid * EPW
    row_base = wid * RPW
    slot_base = s * RPW
    trash = TRASH0 + s

    pltpu.sync_copy(ids_hbm.at[pl.ds(ebase, EPW)], ids_v)
    pltpu.sync_copy(mask_hbm.at[pl.ds(ebase, EPW)], mask_v.at[pl.ds(0, EPW)])

    zeros = jnp.zeros((L,), jnp.float32)
    for i in range(L):
        for j in range(D // L):
            blk_v[i, pl.ds(j * L, L)] = zeros

    def zero_body(g, _):
        pltpu.sync_copy(blk_v, acc_sh.at[pl.ds(slot_base + g * L, L)])
        return 0
    lax.fori_loop(0, RPW // L, zero_body, 0)

    def inv_body(r, _):
        cntv = jnp.zeros((L,), jnp.int32)
        for k in range(H // L):
            mb = mask_v[pl.ds(r * H + k * L, L)] > 0
            cntv = cntv + plsc.all_reduce_population_count(mb)
        tail = mask_v[pl.ds(r * H + (H // L) * L, L)] > 0
        tail = jnp.logical_and(tail, _iota16() < (H % L))
        cntv = cntv + plsc.all_reduce_population_count(tail)
        cf = jnp.maximum(cntv.astype(jnp.float32), 1.0)
        inv_v[pl.ds(r * L, L)] = 1.0 / cf
        return 0
    lax.fori_loop(0, RPW, inv_body, 0)

    def chunk_body(g, _):
        off = g * CH
        pltpu.async_copy(
            table_hbm.at[ids_v.at[pl.ds(off, CH)]], gbuf, sem).wait()
        for j in range(CH // L):
            m = mask_v[pl.ds(off + j * L, L)]
            e = off + j * L + _iota16()
            slot = slot_base + e // H
            dstb[pl.ds(j * L, L)] = jnp.where(m > 0, slot, trash)
        pltpu.sync_copy(gbuf, acc_sh.at[dstb], add=True)
        return 0
    lax.fori_loop(0, NCHUNK, chunk_body, 0)

    def out_body(gb, _):
        pltpu.sync_copy(acc_sh.at[pl.ds(slot_base + gb * L, L)], blk_v)
        for i in range(L):
            inv = inv_v[pl.ds(gb * (L * L) + i * L, L)]
            for j in range(D // L):
                blk_v[i, pl.ds(j * L, L)] = blk_v[i, pl.ds(j * L, L)] * inv
        pltpu.sync_copy(blk_v, out_hbm.at[pl.ds(row_base + gb * L, L)])
        return 0
    lax.fori_loop(0, RPW // L, out_body, 0)


@jax.jit
def _sc_pool(ids_flat, mask_flat, table):
    mesh = plsc.VectorSubcoreMesh(core_axis_name="c", subcore_axis_name="s")
    f = pl.kernel(
        _body,
        out_type=jax.ShapeDtypeStruct((B, D), jnp.float32),
        mesh=mesh,
        compiler_params=pltpu.CompilerParams(needs_layout_passes=False,
                                             use_tc_tiling_on_sc=False),
        scratch_types=[
            pltpu.VMEM((EPW,), jnp.int32),
            pltpu.VMEM((EPW + L,), jnp.int32),
            pltpu.VMEM((CH, D), jnp.float32),
            pltpu.VMEM((CH,), jnp.int32),
            pltpu.VMEM((RPW * L,), jnp.float32),
            pltpu.VMEM((L, D), jnp.float32),
            pltpu.VMEM_SHARED((ACC_ROWS + NS, D), jnp.float32),
            pltpu.SemaphoreType.DMA,
        ],
    )
    return f(ids_flat, mask_flat, table)


def kernel(input_ids, attention_mask, table):
    ids_flat = input_ids.reshape(-1)
    mask_flat = attention_mask.reshape(-1)
    return _sc_pool(ids_flat, mask_flat, table)


# speedup vs baseline: 1.1637x; 1.1637x over previous
import functools

import jax
import jax.numpy as jnp
from jax import lax
from jax.experimental import pallas as pl
from jax.experimental.pallas import tpu as pltpu
from jax.experimental.pallas import tpu_sc as plsc

NC, NS, L = 2, 16, 16
NW = NC * NS
B, H, D = 4096, 200, 64
RPW = B // NW
EPW = RPW * H
CH = 128
NCHUNK = EPW // CH
ACC_ROWS = NS * RPW
TRASH0 = ACC_ROWS


def _iota16():
    return lax.broadcasted_iota(jnp.int32, (L,), 0)


def _body(ids_hbm, mask_hbm, table_hbm, out_hbm,
          ids_v, mask_v, gbuf0, gbuf1, dst0, dst1, inv_v, blk_v, acc_sh,
          sem0, sem1):
    c = lax.axis_index("c")
    s = lax.axis_index("s")
    wid = c * NS + s
    ebase = wid * EPW
    row_base = wid * RPW
    slot_base = s * RPW
    trash = TRASH0 + s

    pltpu.sync_copy(ids_hbm.at[pl.ds(ebase, EPW)], ids_v)
    pltpu.sync_copy(mask_hbm.at[pl.ds(ebase, EPW)], mask_v.at[pl.ds(0, EPW)])

    zeros = jnp.zeros((L,), jnp.float32)
    for i in range(L):
        for j in range(D // L):
            blk_v[i, pl.ds(j * L, L)] = zeros

    def zero_body(g, _):
        pltpu.sync_copy(blk_v, acc_sh.at[pl.ds(slot_base + g * L, L)])
        return 0
    lax.fori_loop(0, RPW // L, zero_body, 0)

    def inv_body(r, _):
        cntv = jnp.zeros((L,), jnp.int32)
        for k in range(H // L):
            mb = mask_v[pl.ds(r * H + k * L, L)] > 0
            cntv = cntv + plsc.all_reduce_population_count(mb)
        tail = mask_v[pl.ds(r * H + (H // L) * L, L)] > 0
        tail = jnp.logical_and(tail, _iota16() < (H % L))
        cntv = cntv + plsc.all_reduce_population_count(tail)
        cf = jnp.maximum(cntv.astype(jnp.float32), 1.0)
        inv_v[pl.ds(r * L, L)] = 1.0 / cf
        return 0
    lax.fori_loop(0, RPW, inv_body, 0)

    bufs = ((gbuf0, dst0, sem0), (gbuf1, dst1, sem1))
    pltpu.async_copy(table_hbm.at[ids_v.at[pl.ds(0, CH)]], gbuf0, sem0)
    pltpu.async_copy(table_hbm.at[ids_v.at[pl.ds(CH, CH)]], gbuf1, sem1)

    def pair_body(i, _):
        for b, (gb, db, sm) in enumerate(bufs):
            g = 2 * i + b
            off = g * CH
            for j in range(CH // L):
                m = mask_v[pl.ds(off + j * L, L)]
                e = off + j * L + _iota16()
                slot = slot_base + e // H
                db[pl.ds(j * L, L)] = jnp.where(m > 0, slot, trash)
            pltpu.make_async_copy(
                table_hbm.at[ids_v.at[pl.ds(off, CH)]], gb, sm).wait()
            pltpu.sync_copy(gb, acc_sh.at[db], add=True)

            @pl.when(g + 2 < NCHUNK)
            def _():
                pltpu.async_copy(
                    table_hbm.at[ids_v.at[pl.ds(off + 2 * CH, CH)]], gb, sm)
        return 0
    lax.fori_loop(0, NCHUNK // 2, pair_body, 0)

    def out_body(gb, _):
        pltpu.sync_copy(acc_sh.at[pl.ds(slot_base + gb * L, L)], blk_v)
        for i in range(L):
            inv = inv_v[pl.ds(gb * (L * L) + i * L, L)]
            for j in range(D // L):
                blk_v[i, pl.ds(j * L, L)] = blk_v[i, pl.ds(j * L, L)] * inv
        pltpu.sync_copy(blk_v, out_hbm.at[pl.ds(row_base + gb * L, L)])
        return 0
    lax.fori_loop(0, RPW // L, out_body, 0)


@jax.jit
def _sc_pool(ids_flat, mask_flat, table):
    mesh = plsc.VectorSubcoreMesh(core_axis_name="c", subcore_axis_name="s")
    f = pl.kernel(
        _body,
        out_type=jax.ShapeDtypeStruct((B, D), jnp.float32),
        mesh=mesh,
        compiler_params=pltpu.CompilerParams(needs_layout_passes=False,
                                             use_tc_tiling_on_sc=False),
        scratch_types=[
            pltpu.VMEM((EPW,), jnp.int32),
            pltpu.VMEM((EPW + L,), jnp.int32),
            pltpu.VMEM((CH, D), jnp.float32),
            pltpu.VMEM((CH, D), jnp.float32),
            pltpu.VMEM((CH,), jnp.int32),
            pltpu.VMEM((CH,), jnp.int32),
            pltpu.VMEM((RPW * L,), jnp.float32),
            pltpu.VMEM((L, D), jnp.float32),
            pltpu.VMEM_SHARED((ACC_ROWS + NS, D), jnp.float32),
            pltpu.SemaphoreType.DMA,
            pltpu.SemaphoreType.DMA,
        ],
    )
    return f(ids_flat, mask_flat, table)


def kernel(input_ids, attention_mask, table):
    ids_flat = input_ids.reshape(-1)
    mask_flat = attention_mask.reshape(-1)
    return _sc_pool(ids_flat, mask_flat, table)


# speedup vs baseline: 1.2018x; 1.0327x over previous
import functools

import jax
import jax.numpy as jnp
from jax import lax
from jax.experimental import pallas as pl
from jax.experimental.pallas import tpu as pltpu
from jax.experimental.pallas import tpu_sc as plsc

NC, NS, L = 2, 16, 16
NW = NC * NS
B, H, D = 4096, 200, 64
RPW = B // NW
EPW = RPW * H
CH = 128
NCHUNK = EPW // CH
NBUF = 8
LA = NBUF - 2
ACC_ROWS = NS * RPW
TRASH0 = ACC_ROWS


def _iota16():
    return lax.broadcasted_iota(jnp.int32, (L,), 0)


def _body(ids_hbm, mask_hbm, table_hbm, out_hbm,
          ids_v, mask_v, inv_v, blk_v, acc_sh, *ring):
    gbufs = ring[:NBUF]
    dsts = ring[NBUF:2 * NBUF]
    sem_g = ring[2 * NBUF:3 * NBUF]
    sem_s = ring[3 * NBUF:4 * NBUF]

    c = lax.axis_index("c")
    s = lax.axis_index("s")
    wid = c * NS + s
    ebase = wid * EPW
    row_base = wid * RPW
    slot_base = s * RPW
    trash = TRASH0 + s

    pltpu.sync_copy(ids_hbm.at[pl.ds(ebase, EPW)], ids_v)
    pltpu.sync_copy(mask_hbm.at[pl.ds(ebase, EPW)], mask_v.at[pl.ds(0, EPW)])

    zeros = jnp.zeros((L,), jnp.float32)
    for i in range(L):
        for j in range(D // L):
            blk_v[i, pl.ds(j * L, L)] = zeros

    def zero_body(g, _):
        pltpu.sync_copy(blk_v, acc_sh.at[pl.ds(slot_base + g * L, L)])
        return 0
    lax.fori_loop(0, RPW // L, zero_body, 0)

    def inv_body(r, _):
        cntv = jnp.zeros((L,), jnp.int32)
        for k in range(H // L):
            mb = mask_v[pl.ds(r * H + k * L, L)] > 0
            cntv = cntv + plsc.all_reduce_population_count(mb)
        tail = mask_v[pl.ds(r * H + (H // L) * L, L)] > 0
        tail = jnp.logical_and(tail, _iota16() < (H % L))
        cntv = cntv + plsc.all_reduce_population_count(tail)
        cf = jnp.maximum(cntv.astype(jnp.float32), 1.0)
        inv_v[pl.ds(r * L, L)] = 1.0 / cf
        return 0
    lax.fori_loop(0, RPW, inv_body, 0)

    def gather_desc(g, b):
        return pltpu.make_async_copy(
            table_hbm.at[ids_v.at[pl.ds(g * CH, CH)]], gbufs[b], sem_g[b])

    def scatter_start(b):
        pltpu.async_copy(gbufs[b], acc_sh.at[dsts[b]], sem_s[b], add=True)

    def scatter_wait(b):
        pltpu.make_async_copy(gbufs[b], acc_sh.at[dsts[b]], sem_s[b]).wait()

    for g0 in range(LA):
        gather_desc(g0, g0).start()

    def ring_body(i, _):
        for b in range(NBUF):
            g = i * NBUF + b
            b2 = (b - 2) % NBUF

            @pl.when(g >= 2)
            def _():
                scatter_wait(b2)

            @pl.when(g + LA < NCHUNK)
            def _():
                gather_desc(g + LA, b2).start()

            off = g * CH
            db = dsts[b]
            for j in range(CH // L):
                m = mask_v[pl.ds(off + j * L, L)]
                e = off + j * L + _iota16()
                slot = slot_base + e // H
                db[pl.ds(j * L, L)] = jnp.where(m > 0, slot, trash)
            gather_desc(g, b).wait()
            scatter_start(b)
        return 0
    lax.fori_loop(0, NCHUNK // NBUF, ring_body, 0)

    scatter_wait((NCHUNK - 2) % NBUF)
    scatter_wait((NCHUNK - 1) % NBUF)

    def out_body(gb, _):
        pltpu.sync_copy(acc_sh.at[pl.ds(slot_base + gb * L, L)], blk_v)
        for i in range(L):
            inv = inv_v[pl.ds(gb * (L * L) + i * L, L)]
            for j in range(D // L):
                blk_v[i, pl.ds(j * L, L)] = blk_v[i, pl.ds(j * L, L)] * inv
        pltpu.sync_copy(blk_v, out_hbm.at[pl.ds(row_base + gb * L, L)])
        return 0
    lax.fori_loop(0, RPW // L, out_body, 0)


@jax.jit
def _sc_pool(ids_flat, mask_flat, table):
    mesh = plsc.VectorSubcoreMesh(core_axis_name="c", subcore_axis_name="s")
    f = pl.kernel(
        _body,
        out_type=jax.ShapeDtypeStruct((B, D), jnp.float32),
        mesh=mesh,
        compiler_params=pltpu.CompilerParams(needs_layout_passes=False,
                                             use_tc_tiling_on_sc=False),
        scratch_types=(
            [
                pltpu.VMEM((EPW,), jnp.int32),
                pltpu.VMEM((EPW + L,), jnp.int32),
                pltpu.VMEM((RPW * L,), jnp.float32),
                pltpu.VMEM((L, D), jnp.float32),
                pltpu.VMEM_SHARED((ACC_ROWS + NS, D), jnp.float32),
            ]
            + [pltpu.VMEM((CH, D), jnp.float32)] * NBUF
            + [pltpu.VMEM((CH,), jnp.int32)] * NBUF
            + [pltpu.SemaphoreType.DMA] * (2 * NBUF)
        ),
    )
    return f(ids_flat, mask_flat, table)


def kernel(input_ids, attention_mask, table):
    ids_flat = input_ids.reshape(-1)
    mask_flat = attention_mask.reshape(-1)
    return _sc_pool(ids_flat, mask_flat, table)
